# baseline (speedup 1.0000x reference)
import functools

import jax
import jax.numpy as jnp
from jax import lax
from jax.experimental import pallas as pl
from jax.experimental.pallas import tpu as pltpu
from jax.experimental.pallas import tpu_sc as plsc

_D = 64
_L = 16
_CH = 128


@functools.lru_cache(maxsize=None)
def _make_sc_kernel(B: int, T: int, V: int):
    info = plsc.get_sparse_core_info()
    nc, ns = info.num_cores, info.num_subcores
    nw = nc * ns
    npw = (B * T) // nw
    nch = npw // _CH
    wpr = T // npw
    assert (B * T) % nw == 0 and npw % _CH == 0 and T % npw == 0

    mesh = plsc.VectorSubcoreMesh(core_axis_name="c", subcore_axis_name="s")

    @functools.partial(
        pl.kernel,
        mesh=mesh,
        compiler_params=pltpu.CompilerParams(use_tc_tiling_on_sc=False),
        out_type=jax.ShapeDtypeStruct((B, T, _D), jnp.float32),
        scratch_types=[
            pltpu.VMEM((nch, _CH), jnp.int32),
            pltpu.VMEM((npw, _D), jnp.float32),
            pltpu.VMEM((npw, _D), jnp.float32),
            pltpu.SemaphoreType.DMA,
        ],
    )
    def sc_kernel(x_hbm, tok_hbm, pos_hbm, out_hbm, idx_v, rows_v, pos_v, sem):
        wid = lax.axis_index("s") * nc + lax.axis_index("c")
        b = wid // wpr
        t0 = (wid % wpr) * npw
        for j in range(nch):
            pltpu.sync_copy(x_hbm.at[b, pl.ds(t0 + j * _CH, _CH)], idx_v.at[j])
        copies = [
            pltpu.async_copy(
                tok_hbm.at[idx_v.at[j]], rows_v.at[pl.ds(j * _CH, _CH)], sem
            )
            for j in range(nch)
        ]
        pltpu.sync_copy(pos_hbm.at[pl.ds(t0, npw)], pos_v)
        for cp in copies:
            cp.wait()

        def add_row(r, carry):
            for c in range(_D // _L):
                sl = pl.ds(c * _L, _L)
                rows_v[r, sl] = rows_v[r, sl] + pos_v[r, sl]
            return carry

        lax.fori_loop(0, npw, add_row, 0)
        pltpu.sync_copy(rows_v, out_hbm.at[b, pl.ds(t0, npw)])

    return sc_kernel


def kernel(x, token_table, position_table):
    B, T = x.shape
    V, D = token_table.shape
    sc_kernel = _make_sc_kernel(B, T, V)
    return sc_kernel(x.astype(jnp.int32), token_table, position_table)


# speedup vs baseline: 2.0150x; 2.0150x over previous
import functools

import jax
import jax.numpy as jnp
from jax import lax
from jax.experimental import pallas as pl
from jax.experimental.pallas import tpu as pltpu
from jax.experimental.pallas import tpu_sc as plsc

_L = 16


@functools.lru_cache(maxsize=None)
def _make_sc_kernel(B: int, T: int, V: int, D: int):
    info = plsc.get_sparse_core_info()
    nc, ns = info.num_cores, info.num_subcores
    nw = nc * ns
    rpw = D // nw
    assert D % nw == 0 and T % _L == 0

    mesh = plsc.VectorSubcoreMesh(core_axis_name="c", subcore_axis_name="s")

    @functools.partial(
        pl.kernel,
        mesh=mesh,
        compiler_params=pltpu.CompilerParams(
            use_tc_tiling_on_sc=True, needs_layout_passes=False
        ),
        out_type=jax.ShapeDtypeStruct((B, D, T), jnp.float32),
        scratch_types=[
            pltpu.VMEM((V,), jnp.float32),
            pltpu.VMEM((B * T,), jnp.int32),
            pltpu.VMEM((T,), jnp.float32),
            pltpu.VMEM((T,), jnp.float32),
        ],
    )
    def sc_kernel(x_hbm, tabT_hbm, posT_hbm, outT_hbm, row_v, idx_v, pos_v, obuf_v):
        wid = lax.axis_index("s") * nc + lax.axis_index("c")
        for b in range(B):
            pltpu.sync_copy(x_hbm.at[b], idx_v.at[pl.ds(b * T, T)])

        def do_row(d, carry):
            pltpu.sync_copy(tabT_hbm.at[d], row_v)
            pltpu.sync_copy(posT_hbm.at[d], pos_v)
            for b in range(B):
                def inner(i, c2):
                    sl = pl.ds(i * _L, _L)
                    ids = idx_v[pl.ds(b * T + i * _L, _L)]
                    vals = plsc.load_gather(row_v, [ids])
                    obuf_v[sl] = vals + pos_v[sl]
                    return c2

                lax.fori_loop(0, T // _L, inner, 0)
                pltpu.sync_copy(obuf_v, outT_hbm.at[b, d])
            return carry

        lax.fori_loop(rpw * wid, rpw * wid + rpw, do_row, 0)

    return sc_kernel


def kernel(x, token_table, position_table):
    B, T = x.shape
    V, D = token_table.shape
    sc_kernel = _make_sc_kernel(B, T, V, D)
    outT = sc_kernel(x.astype(jnp.int32), token_table.T, position_table.T)
    return jnp.transpose(outT, (0, 2, 1))


# speedup vs baseline: 2.1522x; 1.0681x over previous
import functools

import jax
import jax.numpy as jnp
from jax import lax
from jax.experimental import pallas as pl
from jax.experimental.pallas import tpu as pltpu
from jax.experimental.pallas import tpu_sc as plsc

_L = 16


@functools.lru_cache(maxsize=None)
def _make_sc_kernel(B: int, T: int, V: int, D: int):
    info = plsc.get_sparse_core_info()
    nc, ns = info.num_cores, info.num_subcores
    nw = nc * ns
    rpw = D // nw
    assert D % nw == 0 and T % _L == 0

    mesh = plsc.VectorSubcoreMesh(core_axis_name="c", subcore_axis_name="s")

    @functools.partial(
        pl.kernel,
        mesh=mesh,
        compiler_params=pltpu.CompilerParams(
            use_tc_tiling_on_sc=True, needs_layout_passes=False
        ),
        out_type=jax.ShapeDtypeStruct((B, D, T), jnp.float32),
        scratch_types=[
            pltpu.VMEM((V,), jnp.float32),
            pltpu.VMEM((B * T,), jnp.int32),
            pltpu.VMEM((rpw, T), jnp.float32),
            pltpu.VMEM((2, T), jnp.float32),
            pltpu.SemaphoreType.DMA,
            pltpu.SemaphoreType.DMA,
            pltpu.SemaphoreType.DMA,
            pltpu.SemaphoreType.DMA,
        ],
    )
    def sc_kernel(
        x_hbm, tabT_hbm, posT_hbm, outT_hbm, row_v, idx_v, pos_v, obuf_v,
        row_sem, out_sem0, out_sem1, aux_sem,
    ):
        out_sems = [out_sem0, out_sem1]
        wid = lax.axis_index("s") * nc + lax.axis_index("c")
        d0 = rpw * wid
        row_cp = pltpu.async_copy(tabT_hbm.at[d0], row_v, row_sem)
        x_cps = [
            pltpu.async_copy(
                x_hbm.at[b], idx_v.at[pl.ds(b * T, T)], aux_sem
            )
            for b in range(B)
        ]
        pos_cp = pltpu.async_copy(
            posT_hbm.at[pl.ds(d0, rpw)], pos_v, aux_sem
        )
        for cp in x_cps:
            cp.wait()
        pos_cp.wait()
        row_cp.wait()

        unroll = 8
        n_out = 0

        for r in range(rpw):
            d = d0 + r
            for b in range(B):
                slot = n_out % 2

                def inner_b(i, c2, _r=r, _b=b, _slot=slot):
                    base = _b * T
                    for u in range(unroll):
                        off = i * _L * unroll + u * _L
                        ids = idx_v[pl.ds(base + off, _L)]
                        vals = plsc.load_gather(row_v, [ids])
                        obuf_v[_slot, pl.ds(off, _L)] = (
                            vals + pos_v[_r, pl.ds(off, _L)]
                        )
                    return c2

                if n_out >= 2:
                    pltpu.make_async_copy(
                        obuf_v.at[0], outT_hbm.at[0, 0], out_sems[slot]
                    ).wait()
                lax.fori_loop(0, T // (_L * unroll), inner_b, 0)
                pltpu.async_copy(
                    obuf_v.at[slot], outT_hbm.at[b, d], out_sems[slot]
                )
                n_out += 1

            if r + 1 < rpw:
                pltpu.sync_copy(tabT_hbm.at[d0 + r + 1], row_v)

        for s in range(min(n_out, 2)):
            pltpu.make_async_copy(
                obuf_v.at[0], outT_hbm.at[0, 0], out_sems[s]
            ).wait()

    return sc_kernel


def kernel(x, token_table, position_table):
    B, T = x.shape
    V, D = token_table.shape
    sc_kernel = _make_sc_kernel(B, T, V, D)
    outT = sc_kernel(x.astype(jnp.int32), token_table.T, position_table.T)
    return jnp.transpose(outT, (0, 2, 1))


# speedup vs baseline: 2.6544x; 1.2333x over previous
import functools

import jax
import jax.numpy as jnp
from jax import lax
from jax.experimental import pallas as pl
from jax.experimental.pallas import tpu as pltpu
from jax.experimental.pallas import tpu_sc as plsc

_L = 16


@functools.lru_cache(maxsize=None)
def _make_sc_kernel(B: int, T: int, V: int, D: int):
    info = plsc.get_sparse_core_info()
    nc, ns = info.num_cores, info.num_subcores
    nw = nc * ns
    rpw = D // nw
    assert D % nw == 0 and T % _L == 0

    mesh = plsc.VectorSubcoreMesh(core_axis_name="c", subcore_axis_name="s")

    @functools.partial(
        pl.kernel,
        mesh=mesh,
        compiler_params=pltpu.CompilerParams(
            use_tc_tiling_on_sc=True, needs_layout_passes=False
        ),
        out_type=jax.ShapeDtypeStruct((B, D, T), jnp.float32),
        scratch_types=[
            pltpu.VMEM((V,), jnp.float32),
            pltpu.VMEM((B * T,), jnp.int32),
            pltpu.VMEM((rpw, T), jnp.float32),
            pltpu.VMEM((2, T), jnp.float32),
            pltpu.SemaphoreType.DMA,
            pltpu.SemaphoreType.DMA,
            pltpu.SemaphoreType.DMA,
            pltpu.SemaphoreType.DMA,
        ],
    )
    def sc_kernel(
        x_hbm, tabT_hbm, posT_hbm, outT_hbm, row_v, idx_v, pos_v, obuf_v,
        row_sem, out_sem0, out_sem1, aux_sem,
    ):
        out_sems = [out_sem0, out_sem1]
        wid = lax.axis_index("s") * nc + lax.axis_index("c")
        d0 = rpw * wid
        row_cp = pltpu.async_copy(tabT_hbm.at[d0], row_v, row_sem)
        x_cps = [
            pltpu.async_copy(
                x_hbm.at[b], idx_v.at[pl.ds(b * T, T)], aux_sem
            )
            for b in range(B)
        ]
        pos_cp = pltpu.async_copy(
            posT_hbm.at[pl.ds(d0, rpw)], pos_v, aux_sem
        )
        for cp in x_cps:
            cp.wait()
        pos_cp.wait()
        row_cp.wait()

        unroll = 8
        n_out = 0

        for r in range(rpw):
            d = d0 + r
            for b in range(B):
                slot = n_out % 2

                if n_out >= 2:
                    pltpu.make_async_copy(
                        obuf_v.at[0], outT_hbm.at[0, 0], out_sems[slot]
                    ).wait()

                @plsc.parallel_loop(0, T, _L, unroll=unroll)
                def inner_b(off, _r=r, _b=b, _slot=slot):
                    ids = idx_v[pl.ds(_b * T + off, _L)]
                    vals = plsc.load_gather(row_v, [ids])
                    obuf_v[_slot, pl.ds(off, _L)] = (
                        vals + pos_v[_r, pl.ds(off, _L)]
                    )
                pltpu.async_copy(
                    obuf_v.at[slot], outT_hbm.at[b, d], out_sems[slot]
                )
                n_out += 1

            if r + 1 < rpw:
                pltpu.sync_copy(tabT_hbm.at[d0 + r + 1], row_v)

        for s in range(min(n_out, 2)):
            pltpu.make_async_copy(
                obuf_v.at[0], outT_hbm.at[0, 0], out_sems[s]
            ).wait()

    return sc_kernel


def kernel(x, token_table, position_table):
    B, T = x.shape
    V, D = token_table.shape
    sc_kernel = _make_sc_kernel(B, T, V, D)
    outT = sc_kernel(x.astype(jnp.int32), token_table.T, position_table.T)
    return jnp.transpose(outT, (0, 2, 1))
